# baseline (speedup 1.0000x reference)
import jax
import jax.numpy as jnp
import numpy as np
from jax import lax
from jax.experimental import pallas as pl
from jax.experimental.pallas import tpu as pltpu
from jax.experimental.pallas import tpu_sc as plsc

N = 50000
E = 800000
D = 20
FW = 340
FWP = 352
PW = 16
CHUNK = 5000
NPASS = 5
ACC_ROWS = 5120
T = 1000
G = 32
L16 = 16

_RSQRT_MAGIC = np.int32(0x5F3759DF)


def _splat_i32(v):
    return jnp.full((L16,), v, jnp.int32)


def _sc_body(nodetab, postab, edges, aggr,
             acc, colbuf, rowbuf, gcol, grow, lcol, gidx, xg, pg, shm, fbuf, sem):
    ci = lax.axis_index("c")
    si = lax.axis_index("s")
    lanes = lax.iota(jnp.int32, L16)
    zvec = jnp.zeros((L16,), jnp.float32)

    def edge_body(e, carry):
        sh_e = plsc.load_gather(shm, [lanes, _splat_i32(0) + e])
        fbuf[e, pl.ds(0, L16)] = xg[e, pl.ds(0, L16)]
        fbuf[e, pl.ds(L16, L16)] = xg[e, pl.ds(L16, L16)]
        e_splat = _splat_i32(0) + e
        for d_ in range(D):
            fb = plsc.load_gather(xg, [e_splat, _splat_i32(d_)])
            fbuf[e, pl.ds(D + d_ * L16, L16)] = fb * sh_e
        return carry

    def pass_body(p, carry):
        base = (ci * NPASS + p) * CHUNK

        def zrow(r, carry_z):
            for j in range(FWP // L16):
                fbuf[r, pl.ds(j * L16, L16)] = zvec
            return carry_z
        lax.fori_loop(0, G, zrow, 0)
        r0 = si * (ACC_ROWS // 16)
        for j in range(ACC_ROWS // 16 // G):
            pltpu.sync_copy(fbuf, acc.at[pl.ds(r0 + j * G, G)])
        plsc.subcore_barrier()

        ebase = si * (E // 16)

        def tile_body(t, carry_t):
            toff = ebase + t * T
            pltpu.sync_copy(edges.at[0, pl.ds(toff, T)], rowbuf.at[pl.ds(0, T)])
            pltpu.sync_copy(edges.at[1, pl.ds(toff, T)], colbuf.at[pl.ds(0, T)])

            def scan_step(i, k):
                c = colbuf[pl.ds(i * L16, L16)]
                r = rowbuf[pl.ds(i * L16, L16)]
                m = ((c >= base) & (c < base + CHUNK)
                     & (i * L16 + lanes < T))
                mi = m.astype(jnp.int32)
                cs = plsc.cumsum(mi)
                idx = k + cs - 1
                plsc.store_scatter(gcol, [idx], c, mask=m)
                plsc.store_scatter(grow, [idx], r, mask=m)
                return k + cs[L16 - 1]

            k = lax.fori_loop(0, pl.cdiv(T, L16), scan_step, jnp.int32(0))

            for j in range(G // L16):
                plsc.store_scatter(gcol, [k + j * L16 + lanes], _splat_i32(0))
                plsc.store_scatter(grow, [k + j * L16 + lanes], _splat_i32(0))

            ng = (k + (G - 1)) // G

            def group_body(g, carry_g):
                g0 = g * G
                for j in range(G // L16):
                    cg = gcol[pl.ds(g0 + j * L16, L16)]
                    pos = lanes + (g0 + j * L16)
                    lcol[0, pl.ds(j * L16, L16)] = jnp.where(
                        pos < k, cg - base, np.int32(CHUNK))
                    gidx[0, pl.ds(j * L16, L16)] = grow[pl.ds(g0 + j * L16, L16)]
                    gidx[1, pl.ds(j * L16, L16)] = cg
                dsc1 = pltpu.async_copy(nodetab.at[gidx.at[0]], xg, sem)
                dsc2 = pltpu.async_copy(postab.at[gidx.at[1]], pg, sem)
                dsc1.wait()
                dsc2.wait()

                for sub in range(G // L16):
                    eids = lanes + sub * L16
                    px = plsc.load_gather(xg, [eids, _splat_i32(D)])
                    py = plsc.load_gather(xg, [eids, _splat_i32(D + 1)])
                    pz = plsc.load_gather(xg, [eids, _splat_i32(D + 2)])
                    qx = plsc.load_gather(pg, [eids, _splat_i32(0)])
                    qy = plsc.load_gather(pg, [eids, _splat_i32(1)])
                    qz = plsc.load_gather(pg, [eids, _splat_i32(2)])
                    dx = qx - px
                    dy = qy - py
                    dz = qz - pz
                    s = jnp.maximum(dx * dx + dy * dy + dz * dz, 1e-24)
                    yy = plsc.bitcast(
                        _RSQRT_MAGIC - (plsc.bitcast(s, jnp.int32) >> 1),
                        jnp.float32)
                    for _ in range(3):
                        yy = yy * (1.5 - 0.5 * s * yy * yy)
                    ux = dx * yy
                    uy = dy * yy
                    uz = dz * yy
                    x2 = ux * ux
                    y2 = uy * uy
                    z2 = uz * uz
                    sl = pl.ds(sub * L16, L16)
                    shm[0, sl] = jnp.full((L16,), 0.28209479177387814, jnp.float32)
                    shm[1, sl] = 0.4886025119029199 * uy
                    shm[2, sl] = 0.4886025119029199 * uz
                    shm[3, sl] = 0.4886025119029199 * ux
                    shm[4, sl] = 1.0925484305920792 * ux * uy
                    shm[5, sl] = 1.0925484305920792 * uy * uz
                    shm[6, sl] = 0.31539156525252005 * (3.0 * z2 - 1.0)
                    shm[7, sl] = 1.0925484305920792 * ux * uz
                    shm[8, sl] = 0.5462742152960396 * (x2 - y2)
                    shm[9, sl] = 0.5900435899266435 * uy * (3.0 * x2 - y2)
                    shm[10, sl] = 2.890611442640554 * ux * uy * uz
                    shm[11, sl] = 0.4570457994644658 * uy * (5.0 * z2 - 1.0)
                    shm[12, sl] = 0.3731763325901154 * uz * (5.0 * z2 - 3.0)
                    shm[13, sl] = 0.4570457994644658 * ux * (5.0 * z2 - 1.0)
                    shm[14, sl] = 1.445305721320277 * uz * (x2 - y2)
                    shm[15, sl] = 0.5900435899266435 * ux * (3.0 * x2 - y2)

                lax.fori_loop(0, G, edge_body, 0)

                pltpu.sync_copy(fbuf, acc.at[lcol.at[0]], add=True)
                return carry_g

            lax.fori_loop(0, ng, group_body, 0)
            return carry_t

        lax.fori_loop(0, E // 16 // T, tile_body, 0)
        plsc.subcore_barrier()

        w0 = si * 312
        pltpu.sync_copy(acc.at[pl.ds(w0, 312)], aggr.at[pl.ds(base + w0, 312)])

        @pl.when(si == 0)
        def _copy_tail():
            pltpu.sync_copy(acc.at[pl.ds(4992, 8)], aggr.at[pl.ds(base + 4992, 8)])

        plsc.subcore_barrier()
        return carry

    lax.fori_loop(0, NPASS, pass_body, 0)


def _sc_aggregate(nodetab, postab, edge_index):
    mesh = plsc.VectorSubcoreMesh(core_axis_name="c", subcore_axis_name="s")
    return pl.kernel(
        _sc_body,
        out_type=jax.ShapeDtypeStruct((N, FWP), jnp.float32),
        mesh=mesh,
        compiler_params=pltpu.CompilerParams(
            use_tc_tiling_on_sc=False, needs_layout_passes=False),
        scratch_types=[
            pltpu.VMEM_SHARED((ACC_ROWS, FWP), jnp.float32),
            pltpu.VMEM((T + L16,), jnp.int32),
            pltpu.VMEM((T + L16,), jnp.int32),
            pltpu.VMEM((T + 2 * G,), jnp.int32),
            pltpu.VMEM((T + 2 * G,), jnp.int32),
            pltpu.VMEM((1, G), jnp.int32),
            pltpu.VMEM((2, G), jnp.int32),
            pltpu.VMEM((G, 32), jnp.float32),
            pltpu.VMEM((G, PW), jnp.float32),
            pltpu.VMEM((L16, G), jnp.float32),
            pltpu.VMEM((G, FWP), jnp.float32),
            pltpu.SemaphoreType.DMA,
        ],
    )(nodetab, postab, edge_index)


def _mlp_body(x_ref, wpre_ref, bpre_ref, wpost_ref, bpost_ref, wsc_ref, bsc_ref, o_ref):
    x = x_ref[...]
    h = jnp.maximum(jnp.dot(x, wpre_ref[...], preferred_element_type=jnp.float32)
                    + bpre_ref[...], 0.0)
    h = jnp.dot(h, wpost_ref[...], preferred_element_type=jnp.float32) + bpost_ref[...]
    o_ref[...] = h + jnp.dot(x, wsc_ref[...], preferred_element_type=jnp.float32) + bsc_ref[...]


def kernel(x, edge_index, positions, W_pre, b_pre, W_post, b_post, W_sc, b_sc):
    n, d = x.shape
    B = 5000
    out = pl.pallas_call(
        _mlp_body,
        grid=(n // B,),
        in_specs=[
            pl.BlockSpec((B, d), lambda i: (i, 0)),
            pl.BlockSpec((d, d), lambda i: (0, 0)),
            pl.BlockSpec((1, d), lambda i: (0, 0)),
            pl.BlockSpec((d, d), lambda i: (0, 0)),
            pl.BlockSpec((1, d), lambda i: (0, 0)),
            pl.BlockSpec((d, d), lambda i: (0, 0)),
            pl.BlockSpec((1, d), lambda i: (0, 0)),
        ],
        out_specs=pl.BlockSpec((B, d), lambda i: (i, 0)),
        out_shape=jax.ShapeDtypeStruct((n, d), jnp.float32),
    )(x, W_pre, b_pre.reshape(1, d), W_post, b_post.reshape(1, d),
      W_sc, b_sc.reshape(1, d))

    nodetab = jnp.concatenate(
        [x, positions, jnp.zeros((n, 32 - d - 3), jnp.float32)], axis=1)
    postab = jnp.concatenate(
        [positions, jnp.zeros((n, PW - 3), jnp.float32)], axis=1)
    aggr = _sc_aggregate(nodetab, postab, edge_index)
    return out, aggr[:, :FW]


# speedup vs baseline: 1.3689x; 1.3689x over previous
import jax
import jax.numpy as jnp
import numpy as np
from jax import lax
from jax.experimental import pallas as pl
from jax.experimental.pallas import tpu as pltpu
from jax.experimental.pallas import tpu_sc as plsc

N = 50000
E = 800000
D = 20
FW = 340
FWP = 352
PW = 16
CHUNK = 5000
NPASS = 5
ACC_ROWS = 5120
T = 1000
G = 32
L16 = 16

_RSQRT_MAGIC = np.int32(0x5F3759DF)


def _splat_i32(v):
    return jnp.full((L16,), v, jnp.int32)


def _sc_body(nodetab, postab, edges, aggr,
             acc, colbuf, rowbuf, gcol, grow, lcol, gidx, xg, pg, shm, fbuf,
             sem_i, sem_g, sem_s):
    ci = lax.axis_index("c")
    si = lax.axis_index("s")
    lanes = lax.iota(jnp.int32, L16)
    zvec = jnp.zeros((L16,), jnp.float32)
    NT = E // 16 // T

    def edge_body(par):
        def _eb(e, carry):
            sh_e = plsc.load_gather(shm, [lanes, _splat_i32(0) + e])
            fbuf[par, e, pl.ds(0, L16)] = xg[par, e, pl.ds(0, L16)]
            fbuf[par, e, pl.ds(L16, L16)] = xg[par, e, pl.ds(L16, L16)]
            p_splat = _splat_i32(0) + par
            e_splat = _splat_i32(0) + e
            for d_ in range(D):
                fb = plsc.load_gather(xg, [p_splat, e_splat, _splat_i32(d_)])
                fbuf[par, e, pl.ds(D + d_ * L16, L16)] = fb * sh_e
            return carry
        return _eb

    def pass_body(p, carry):
        base = (ci * NPASS + p) * CHUNK

        def zrow(r, carry_z):
            for j in range(FWP // L16):
                fbuf[0, r, pl.ds(j * L16, L16)] = zvec
            return carry_z
        lax.fori_loop(0, L16, zrow, 0)
        r0 = si * (ACC_ROWS // 16)
        for j in range(ACC_ROWS // 16 // L16):
            pltpu.sync_copy(fbuf.at[0], acc.at[pl.ds(r0 + j * L16, L16)])
        plsc.subcore_barrier()

        ebase = si * (E // 16)

        pltpu.async_copy(edges.at[0, pl.ds(ebase, T)], rowbuf.at[pl.ds(0, T)], sem_i)
        pltpu.async_copy(edges.at[1, pl.ds(ebase, T)], colbuf.at[pl.ds(0, T)], sem_i)

        def tile_body(t, carry_t):
            toff = ebase + t * T
            pltpu.make_async_copy(edges.at[0, pl.ds(toff, T)],
                                  rowbuf.at[pl.ds(0, T)], sem_i).wait()
            pltpu.make_async_copy(edges.at[1, pl.ds(toff, T)],
                                  colbuf.at[pl.ds(0, T)], sem_i).wait()

            def scan_step(i, k):
                c = colbuf[pl.ds(i * L16, L16)]
                r = rowbuf[pl.ds(i * L16, L16)]
                m = ((c >= base) & (c < base + CHUNK)
                     & (i * L16 + lanes < T))
                mi = m.astype(jnp.int32)
                cs = plsc.cumsum(mi)
                idx = k + cs - 1
                plsc.store_scatter(gcol, [idx], c, mask=m)
                plsc.store_scatter(grow, [idx], r, mask=m)
                return k + cs[L16 - 1]

            k = lax.fori_loop(0, pl.cdiv(T, L16), scan_step, jnp.int32(0))

            plsc.store_scatter(gcol, [k + lanes], _splat_i32(0))
            plsc.store_scatter(grow, [k + lanes], _splat_i32(0))

            @pl.when(t + 1 < NT)
            def _prefetch_idx():
                toff2 = ebase + (t + 1) * T
                pltpu.async_copy(edges.at[0, pl.ds(toff2, T)],
                                 rowbuf.at[pl.ds(0, T)], sem_i)
                pltpu.async_copy(edges.at[1, pl.ds(toff2, T)],
                                 colbuf.at[pl.ds(0, T)], sem_i)

            nh = (k + (L16 - 1)) // L16

            @pl.when(nh > 0)
            def _prime():
                gidx[0, 0, pl.ds(0, L16)] = grow[pl.ds(0, L16)]
                gidx[0, 1, pl.ds(0, L16)] = gcol[pl.ds(0, L16)]
                pltpu.async_copy(nodetab.at[gidx.at[0, 0]], xg.at[0], sem_g)
                pltpu.async_copy(postab.at[gidx.at[0, 1]], pg.at[0], sem_g)

            def hbody(h, carry_h):
                par = h % 2
                opar = 1 - par
                pltpu.make_async_copy(nodetab.at[gidx.at[par, 0]],
                                      xg.at[par], sem_g).wait()
                pltpu.make_async_copy(postab.at[gidx.at[par, 1]],
                                      pg.at[par], sem_g).wait()

                @pl.when(h + 1 < nh)
                def _prefetch_g():
                    g1 = (h + 1) * L16
                    gidx[opar, 0, pl.ds(0, L16)] = grow[pl.ds(g1, L16)]
                    gidx[opar, 1, pl.ds(0, L16)] = gcol[pl.ds(g1, L16)]
                    pltpu.async_copy(nodetab.at[gidx.at[opar, 0]], xg.at[opar], sem_g)
                    pltpu.async_copy(postab.at[gidx.at[opar, 1]], pg.at[opar], sem_g)

                @pl.when(h >= 2)
                def _drain():
                    pltpu.make_async_copy(fbuf.at[par], acc.at[lcol.at[par]],
                                          sem_s).wait()

                p_splat = _splat_i32(0) + par
                px = plsc.load_gather(xg, [p_splat, lanes, _splat_i32(D)])
                py = plsc.load_gather(xg, [p_splat, lanes, _splat_i32(D + 1)])
                pz = plsc.load_gather(xg, [p_splat, lanes, _splat_i32(D + 2)])
                qx = plsc.load_gather(pg, [p_splat, lanes, _splat_i32(0)])
                qy = plsc.load_gather(pg, [p_splat, lanes, _splat_i32(1)])
                qz = plsc.load_gather(pg, [p_splat, lanes, _splat_i32(2)])
                dx = qx - px
                dy = qy - py
                dz = qz - pz
                s = jnp.maximum(dx * dx + dy * dy + dz * dz, 1e-24)
                yy = plsc.bitcast(
                    _RSQRT_MAGIC - (plsc.bitcast(s, jnp.int32) >> 1),
                    jnp.float32)
                for _ in range(3):
                    yy = yy * (1.5 - 0.5 * s * yy * yy)
                ux = dx * yy
                uy = dy * yy
                uz = dz * yy
                x2 = ux * ux
                y2 = uy * uy
                z2 = uz * uz
                sl = pl.ds(0, L16)
                shm[0, sl] = jnp.full((L16,), 0.28209479177387814, jnp.float32)
                shm[1, sl] = 0.4886025119029199 * uy
                shm[2, sl] = 0.4886025119029199 * uz
                shm[3, sl] = 0.4886025119029199 * ux
                shm[4, sl] = 1.0925484305920792 * ux * uy
                shm[5, sl] = 1.0925484305920792 * uy * uz
                shm[6, sl] = 0.31539156525252005 * (3.0 * z2 - 1.0)
                shm[7, sl] = 1.0925484305920792 * ux * uz
                shm[8, sl] = 0.5462742152960396 * (x2 - y2)
                shm[9, sl] = 0.5900435899266435 * uy * (3.0 * x2 - y2)
                shm[10, sl] = 2.890611442640554 * ux * uy * uz
                shm[11, sl] = 0.4570457994644658 * uy * (5.0 * z2 - 1.0)
                shm[12, sl] = 0.3731763325901154 * uz * (5.0 * z2 - 3.0)
                shm[13, sl] = 0.4570457994644658 * ux * (5.0 * z2 - 1.0)
                shm[14, sl] = 1.445305721320277 * uz * (x2 - y2)
                shm[15, sl] = 0.5900435899266435 * ux * (3.0 * x2 - y2)

                cg = gidx[par, 1, pl.ds(0, L16)]
                pos = lanes + h * L16
                lcol[par, pl.ds(0, L16)] = jnp.where(
                    pos < k, cg - base, np.int32(CHUNK))

                lax.fori_loop(0, L16, edge_body(par), 0)

                pltpu.async_copy(fbuf.at[par], acc.at[lcol.at[par]],
                                 sem_s, add=True)
                return carry_h

            lax.fori_loop(0, nh, hbody, 0)

            @pl.when(nh >= 1)
            def _drain1():
                pltpu.make_async_copy(fbuf.at[0], acc.at[lcol.at[0]],
                                      sem_s).wait()

            @pl.when(nh >= 2)
            def _drain2():
                pltpu.make_async_copy(fbuf.at[1], acc.at[lcol.at[1]],
                                      sem_s).wait()
            return carry_t

        lax.fori_loop(0, NT, tile_body, 0)
        plsc.subcore_barrier()

        w0 = si * 312
        pltpu.sync_copy(acc.at[pl.ds(w0, 312)], aggr.at[pl.ds(base + w0, 312)])

        @pl.when(si == 0)
        def _copy_tail():
            pltpu.sync_copy(acc.at[pl.ds(4992, 8)], aggr.at[pl.ds(base + 4992, 8)])

        plsc.subcore_barrier()
        return carry

    lax.fori_loop(0, NPASS, pass_body, 0)


def _sc_aggregate(nodetab, postab, edge_index):
    mesh = plsc.VectorSubcoreMesh(core_axis_name="c", subcore_axis_name="s")
    return pl.kernel(
        _sc_body,
        out_type=jax.ShapeDtypeStruct((N, FWP), jnp.float32),
        mesh=mesh,
        compiler_params=pltpu.CompilerParams(
            use_tc_tiling_on_sc=False, needs_layout_passes=False),
        scratch_types=[
            pltpu.VMEM_SHARED((ACC_ROWS, FWP), jnp.float32),
            pltpu.VMEM((T + L16,), jnp.int32),
            pltpu.VMEM((T + L16,), jnp.int32),
            pltpu.VMEM((T + 2 * L16,), jnp.int32),
            pltpu.VMEM((T + 2 * L16,), jnp.int32),
            pltpu.VMEM((2, L16), jnp.int32),
            pltpu.VMEM((2, 2, L16), jnp.int32),
            pltpu.VMEM((2, L16, 32), jnp.float32),
            pltpu.VMEM((2, L16, PW), jnp.float32),
            pltpu.VMEM((L16, L16), jnp.float32),
            pltpu.VMEM((2, L16, FWP), jnp.float32),
            pltpu.SemaphoreType.DMA,
            pltpu.SemaphoreType.DMA,
            pltpu.SemaphoreType.DMA,
        ],
    )(nodetab, postab, edge_index)


def _mlp_body(x_ref, wpre_ref, bpre_ref, wpost_ref, bpost_ref, wsc_ref, bsc_ref, o_ref):
    x = x_ref[...]
    h = jnp.maximum(jnp.dot(x, wpre_ref[...], preferred_element_type=jnp.float32)
                    + bpre_ref[...], 0.0)
    h = jnp.dot(h, wpost_ref[...], preferred_element_type=jnp.float32) + bpost_ref[...]
    o_ref[...] = h + jnp.dot(x, wsc_ref[...], preferred_element_type=jnp.float32) + bsc_ref[...]


def kernel(x, edge_index, positions, W_pre, b_pre, W_post, b_post, W_sc, b_sc):
    n, d = x.shape
    B = 5000
    out = pl.pallas_call(
        _mlp_body,
        grid=(n // B,),
        in_specs=[
            pl.BlockSpec((B, d), lambda i: (i, 0)),
            pl.BlockSpec((d, d), lambda i: (0, 0)),
            pl.BlockSpec((1, d), lambda i: (0, 0)),
            pl.BlockSpec((d, d), lambda i: (0, 0)),
            pl.BlockSpec((1, d), lambda i: (0, 0)),
            pl.BlockSpec((d, d), lambda i: (0, 0)),
            pl.BlockSpec((1, d), lambda i: (0, 0)),
        ],
        out_specs=pl.BlockSpec((B, d), lambda i: (i, 0)),
        out_shape=jax.ShapeDtypeStruct((n, d), jnp.float32),
    )(x, W_pre, b_pre.reshape(1, d), W_post, b_post.reshape(1, d),
      W_sc, b_sc.reshape(1, d))

    nodetab = jnp.concatenate(
        [x, positions, jnp.zeros((n, 32 - d - 3), jnp.float32)], axis=1)
    postab = jnp.concatenate(
        [positions, jnp.zeros((n, PW - 3), jnp.float32)], axis=1)
    aggr = _sc_aggregate(nodetab, postab, edge_index)
    return out, aggr[:, :FW]
